# baseline (speedup 1.0000x reference)
import functools

import jax
import jax.numpy as jnp
from jax import lax
from jax.experimental import pallas as pl
from jax.experimental.pallas import tpu as pltpu
from jax.experimental.pallas import tpu_sc as plsc

NC = 2
NS = 16
K = 80
NB = 4
R0 = 624


def _when(cond, fn):
    if isinstance(cond, bool):
        if cond:
            fn()
    else:
        pl.when(cond)(fn)


def _shard_rows(n, s, do):
    do(s * R0, R0)

    @pl.when(s == NS - 1)
    def _():
        do(NS * R0, n - NS * R0)


def _mesh():
    return plsc.VectorSubcoreMesh(
        core_axis_name="c", subcore_axis_name="s",
        num_cores=NC, num_subcores=NS)


def _make_pp_kernel(n, e):
    epc = e // (NC * NS)
    nchunks = epc // K

    @functools.partial(
        pl.kernel,
        out_type=jax.ShapeDtypeStruct((2 * n, 16), jnp.float32),
        mesh=_mesh(),
        compiler_params=pltpu.CompilerParams(needs_layout_passes=False,
                                             use_tc_tiling_on_sc=False),
        scratch_types=(
            [pltpu.VMEM((K,), jnp.int32) for _ in range(NB)]
            + [pltpu.VMEM((K, 16), jnp.float32) for _ in range(NB)]
            + [pltpu.VMEM_SHARED((n + 1, 16), jnp.float32)]
            + [pltpu.SemaphoreType.DMA for _ in range(2 * NB)]
        ),
    )
    def pp(dstm_h, eamf_h, z_h, out_h, *scr):
        dstmb = scr[0:NB]
        rowb = scr[NB:2 * NB]
        tab_sh = scr[2 * NB]
        s_ix = scr[2 * NB + 1:2 * NB + 1 + NB]
        s_sc = scr[2 * NB + 1 + NB:2 * NB + 1 + 2 * NB]
        c = lax.axis_index("c")
        s = lax.axis_index("s")
        _shard_rows(n, s, lambda o, l: pltpu.sync_copy(
            z_h.at[pl.ds(o, l)], tab_sh.at[pl.ds(o, l)]))
        plsc.subcore_barrier()
        base0 = (c * NS + s) * epc

        def issue_ix(i, b):
            pltpu.async_copy(dstm_h.at[pl.ds(base0 + i * K, K)], dstmb[b],
                             s_ix[b])
            pltpu.async_copy(eamf_h.at[pl.ds(base0 + i * K, K)], rowb[b],
                             s_ix[b])

        def drain_sc(b):
            pltpu.make_async_copy(rowb[b], tab_sh.at[dstmb[b]],
                                  s_sc[b]).wait()

        def body(i, b):
            p2 = (b + 2) % NB
            pltpu.make_async_copy(dstm_h.at[pl.ds(0, K)], dstmb[b],
                                  s_ix[b]).wait()
            pltpu.make_async_copy(eamf_h.at[pl.ds(0, K)], rowb[b],
                                  s_ix[b]).wait()
            pltpu.async_copy(rowb[b], tab_sh.at[dstmb[b]], s_sc[b], add=True)

            def _pref():
                _when(i >= 2, lambda: drain_sc(p2))
                issue_ix(i + 2, p2)
            _when(i + 2 < nchunks, _pref)

        issue_ix(0, 0)
        issue_ix(1, 1)

        def outer(g, carry):
            for t in range(NB):
                body(NB * g + t, t)
            return carry
        lax.fori_loop(0, nchunks // NB, outer, None)
        for t in range(nchunks % NB):
            body(nchunks - nchunks % NB + t, t)
        for j in range(max(nchunks - 4, 0), nchunks):
            drain_sc(j % NB)
        plsc.subcore_barrier()
        _shard_rows(n, s, lambda o, l: pltpu.sync_copy(
            tab_sh.at[pl.ds(o, l)], out_h.at[pl.ds(c * n + o, l)]))

    return pp


def _make_edge_kernel(n, e, hpc, w, split_edges):
    ev = 16 // hpc
    epc = e // (NC * NS) if split_edges else e // NS
    nchunks = epc // K
    UN = 12

    @functools.partial(
        pl.kernel,
        out_type=(jax.ShapeDtypeStruct((2 * n, w), jnp.float32),
                  jax.ShapeDtypeStruct((2 * n, 16), jnp.float32)),
        mesh=_mesh(),
        compiler_params=pltpu.CompilerParams(needs_layout_passes=False,
                                             use_tc_tiling_on_sc=False),
        scratch_types=(
            [pltpu.VMEM((K,), jnp.int32) for _ in range(3)]
            + [pltpu.VMEM((K,), jnp.int32) for _ in range(4)]
            + [pltpu.VMEM((K,), jnp.int32) for _ in range(3)]
            + [pltpu.VMEM((K, 16), jnp.float32) for _ in range(2)]
            + [pltpu.VMEM((K, 16), jnp.float32) for _ in range(2)]
            + [pltpu.VMEM((K, 16), jnp.float32) for _ in range(2)]
            + [pltpu.VMEM((K, 8), jnp.float32) for _ in range(2)]
            + [pltpu.VMEM((K, w), jnp.float32) for _ in range(3)]
            + [pltpu.VMEM_SHARED((n + 1, w), jnp.float32),
               pltpu.VMEM_SHARED((n + 1, 16), jnp.float32)]
            + [pltpu.SemaphoreType.DMA for _ in range(4 + 3 + 2)]
        ),
    )
    def gat(src_h, dstm_h, aed_h, al_h, arm_h, hcat_h, inum_h, iden_h,
            num_o, den_o, *scr):
        srcb = scr[0:3]
        dstmb = scr[3:7]
        soffb = scr[7:10]
        bufA = scr[10:12]
        bufB = scr[12:14]
        pb = scr[14:16]
        aedb = scr[16:18]
        hb = scr[18:21]
        num_sh = scr[21]
        den_sh = scr[22]
        s_ix = scr[23:27]
        s_g = scr[27:30]
        s_sc = scr[30:32]

        c = lax.axis_index("c")
        s = lax.axis_index("s")

        def _init(o, l):
            pltpu.sync_copy(inum_h.at[pl.ds(c * n + o, l)],
                            num_sh.at[pl.ds(o, l)])
            pltpu.sync_copy(iden_h.at[pl.ds(c * n + o, l)],
                            den_sh.at[pl.ds(o, l)])
        _shard_rows(n, s, _init)
        plsc.subcore_barrier()
        base0 = (c * NS + s) * epc if split_edges else s * epc
        hoff = 0 if split_edges else c * n
        lane = lax.iota(jnp.int32, 16)
        erep = lax.shift_right_logical(lane, {1: 4, 2: 3, 4: 2, 8: 1, 16: 0}[ev])
        colidx = lax.rem(lane, hpc)

        def issue_ix(ti, t):
            pltpu.async_copy(src_h.at[pl.ds(base0 + ti * K, K)], srcb[t % 3],
                             s_ix[t % 4])
            pltpu.async_copy(dstm_h.at[pl.ds(base0 + ti * K, K)], dstmb[t % 4],
                             s_ix[t % 4])

        def issue_g(ti, t):
            pltpu.make_async_copy(src_h.at[pl.ds(0, K)], srcb[t % 3],
                                  s_ix[t % 4]).wait()
            pltpu.make_async_copy(dstm_h.at[pl.ds(0, K)], dstmb[t % 4],
                                  s_ix[t % 4]).wait()
            pltpu.async_copy(aed_h.at[pl.ds(base0 + ti * K, K)], aedb[t % 2],
                             s_g[t % 3])
            if split_edges:
                pltpu.async_copy(hcat_h.at[srcb[t % 3]], hb[t % 3], s_g[t % 3])
            else:
                def add16(v, cc):
                    sl = pl.ds(v * 16, 16)
                    soffb[t % 3][sl] = srcb[t % 3][sl] + hoff
                    return cc
                lax.fori_loop(0, K // 16, add16, None)
                pltpu.async_copy(hcat_h.at[soffb[t % 3]], hb[t % 3], s_g[t % 3])
            pltpu.async_copy(al_h.at[srcb[t % 3]], bufA[t % 2], s_g[t % 3])
            pltpu.async_copy(arm_h.at[dstmb[t % 4]], bufB[t % 2], s_g[t % 3])

        def drain_sc(t):
            pltpu.make_async_copy(pb[t % 2], den_sh.at[dstmb[t % 4]],
                                  s_sc[t % 2]).wait()
            pltpu.make_async_copy(hb[t % 3], num_sh.at[dstmb[t % 4]],
                                  s_sc[t % 2]).wait()

        def body(i, t):
            _when(i >= 2, lambda: drain_sc(t - 2))
            _when(i + 2 < nchunks, lambda: issue_ix(i + 2, t + 2))
            _when(i + 1 < nchunks, lambda: issue_g(i + 1, t + 1))

            pltpu.make_async_copy(hcat_h.at[srcb[t % 3]], hb[t % 3],
                                  s_g[t % 3]).wait()
            pltpu.make_async_copy(al_h.at[srcb[t % 3]], bufA[t % 2],
                                  s_g[t % 3]).wait()
            pltpu.make_async_copy(arm_h.at[dstmb[t % 4]], bufB[t % 2],
                                  s_g[t % 3]).wait()
            pltpu.make_async_copy(aed_h.at[pl.ds(0, K)], aedb[t % 2],
                                  s_g[t % 3]).wait()

            col = colidx if split_edges else 4 * c + colidx

            def pcomp(v, cc):
                rloc = erep + v * ev
                aA = plsc.load_gather(bufA[t % 2], [rloc, col])
                aB = plsc.load_gather(bufB[t % 2], [rloc, col])
                aE = plsc.load_gather(aedb[t % 2], [rloc, col])
                a = aA + aB + aE
                a = jnp.where(a >= 0.0, a, 0.2 * a)
                plsc.store_scatter(pb[t % 2], [rloc, colidx], jnp.exp(a))
                return cc
            lax.fori_loop(0, K // ev, pcomp, None)

            def mul(j, cc):
                jj = jnp.full((16,), j, jnp.int32)
                for hh in range(hpc):
                    colh = jnp.full((16,), hh, jnp.int32)
                    sp = plsc.load_gather(pb[t % 2], [jj, colh])
                    for qq in range(w // (16 * hpc)):
                        o = hh * (w // hpc) + qq * 16
                        hb[t % 3][j, pl.ds(o, 16)] = (
                            hb[t % 3][j, pl.ds(o, 16)] * sp)
                return cc
            lax.fori_loop(0, K, mul, None)

            pltpu.async_copy(pb[t % 2], den_sh.at[dstmb[t % 4]],
                             s_sc[t % 2], add=True)
            pltpu.async_copy(hb[t % 3], num_sh.at[dstmb[t % 4]],
                             s_sc[t % 2], add=True)

        issue_ix(0, 0)
        issue_ix(1, 1)
        issue_g(0, 0)

        def outer(g, carry):
            for t in range(UN):
                body(UN * g + t, t)
            return carry
        lax.fori_loop(0, nchunks // UN, outer, None)
        for t in range(nchunks % UN):
            body(nchunks - nchunks % UN + t, t)
        drain_sc(nchunks - 2)
        drain_sc(nchunks - 1)
        plsc.subcore_barrier()

        def _out(o, l):
            pltpu.sync_copy(num_sh.at[pl.ds(o, l)],
                            num_o.at[pl.ds(c * n + o, l)])
            pltpu.sync_copy(den_sh.at[pl.ds(o, l)],
                            den_o.at[pl.ds(c * n + o, l)])
        _shard_rows(n, s, _out)

    return gat


def _finalize_body(num_ref, den_ref, b_ref, o_ref, *, relu):
    v = num_ref[...] / (den_ref[...] + 1e-16) + b_ref[...]
    o_ref[...] = jnp.maximum(v, 0.0) if relu else v


def _finalize(num, den_full, b, relu):
    n, d = num.shape
    blk = 2000
    return pl.pallas_call(
        functools.partial(_finalize_body, relu=relu),
        grid=(n // blk,),
        in_specs=[
            pl.BlockSpec((blk, d), lambda i: (i, 0)),
            pl.BlockSpec((blk, d), lambda i: (i, 0)),
            pl.BlockSpec((1, d), lambda i: (0, 0)),
        ],
        out_specs=pl.BlockSpec((blk, d), lambda i: (i, 0)),
        out_shape=jax.ShapeDtypeStruct((n, d), num.dtype),
    )(num, den_full, b.reshape(1, d))


def _lrelu(a):
    return jnp.where(a >= 0, a, 0.2 * a)


def kernel(x, edge_index, edge_attr, W_se, b_se, W1, att_src1, att_dst1,
           W_e1, att_e1, b1, W2, att_src2, att_dst2, W_e2, att_e2, b2):
    n, d_in = x.shape
    heads, ch = att_src1.shape
    e = edge_index.shape[1]
    src, dst = edge_index[0], edge_index[1]
    dstm = jnp.where(src == dst, n, dst)

    ea = jax.nn.relu(edge_attr @ W_se + b_se)
    eamf = jnp.concatenate(
        [ea, jnp.ones((e, 1), jnp.float32), jnp.zeros((e, 7), jnp.float32)],
        axis=1)
    tab2 = _make_pp_kernel(n, e)(dstm, eamf, jnp.zeros((n, 16), jnp.float32))
    tab = tab2[:n] + tab2[n:]
    deg = tab[:, 8]
    loop_ea = tab[:, :8] / jnp.clip(deg, 1.0)[:, None]

    w1r = W1.reshape(d_in, heads, ch)
    ws1 = jnp.einsum('dhc,hc->dh', w1r, att_src1)
    wd1 = jnp.einsum('dhc,hc->dh', w1r, att_dst1)
    we1 = jnp.einsum('dhc,hc->dh', W_e1.reshape(-1, heads, ch), att_e1)
    h = x @ W1
    al = x @ ws1
    ar = x @ wd1
    aed_e = ea @ we1
    aed_n = loop_ea @ we1
    alA = jnp.tile(al, (1, 2))
    arm = jnp.concatenate(
        [jnp.tile(ar, (1, 2)), jnp.full((1, 16), -1e5, jnp.float32)], axis=0)
    hcat = jnp.concatenate([h[:, :128], h[:, 128:]], axis=0)
    p_self = jnp.exp(_lrelu(al + ar + aed_n))
    inum = (p_self[:, :, None] * h.reshape(n, heads, ch)).reshape(n, 256)
    inum = jnp.concatenate([inum[:, :128], inum[:, 128:]], axis=0)
    zpad = jnp.zeros((n, 12), jnp.float32)
    iden = jnp.concatenate(
        [jnp.concatenate([p_self[:, :4], zpad], axis=1),
         jnp.concatenate([p_self[:, 4:], zpad], axis=1)], axis=0)

    num_o, den_o = _make_edge_kernel(n, e, 4, 128, False)(
        src, dstm, aed_e, alA, arm, hcat, inum, iden)
    num1 = jnp.concatenate([num_o[:n], num_o[n:]], axis=1)
    den1 = jnp.concatenate([den_o[:n, :4], den_o[n:, :4]], axis=1)
    h1 = _finalize(num1, jnp.repeat(den1, ch, axis=1), b1, relu=True)

    ws2 = W2 @ att_src2[0]
    wd2 = W2 @ att_dst2[0]
    we2 = W_e2 @ att_e2[0]
    h2 = h1 @ W2
    al2 = h1 @ ws2
    ar2 = h1 @ wd2
    aed2_e = jnp.pad((ea @ we2)[:, None], ((0, 0), (0, 7)))
    aed2_n = loop_ea @ we2
    alA2 = jnp.pad(al2[:, None], ((0, 0), (0, 15)))
    arm2 = jnp.concatenate(
        [jnp.pad(ar2[:, None], ((0, 0), (0, 15))),
         jnp.full((1, 16), -1e5, jnp.float32)], axis=0)
    p2s = jnp.exp(_lrelu(al2 + ar2 + aed2_n))
    inum2 = jnp.concatenate(
        [p2s[:, None] * h2, jnp.zeros((n, 32), jnp.float32)], axis=0)
    iden2 = jnp.concatenate(
        [jnp.pad(p2s[:, None], ((0, 0), (0, 15))),
         jnp.zeros((n, 16), jnp.float32)], axis=0)

    num_o2, den_o2 = _make_edge_kernel(n, e, 1, 32, True)(
        src, dstm, aed2_e, alA2, arm2, h2, inum2, iden2)
    num2 = num_o2[:n] + num_o2[n:]
    den2 = den_o2[:n, :1] + den_o2[n:, :1]
    out = _finalize(num2, jnp.tile(den2, (1, 32)), b2, relu=False)
    return out


# speedup vs baseline: 1.0148x; 1.0148x over previous
import functools

import jax
import jax.numpy as jnp
from jax import lax
from jax.experimental import pallas as pl
from jax.experimental.pallas import tpu as pltpu
from jax.experimental.pallas import tpu_sc as plsc

NC = 2
NS = 16
K = 80
NB = 4
R0 = 624


def _when(cond, fn):
    if isinstance(cond, bool):
        if cond:
            fn()
    else:
        pl.when(cond)(fn)


def _shard_rows(n, s, do):
    do(s * R0, R0)

    @pl.when(s == NS - 1)
    def _():
        do(NS * R0, n - NS * R0)


def _mesh():
    return plsc.VectorSubcoreMesh(
        core_axis_name="c", subcore_axis_name="s",
        num_cores=NC, num_subcores=NS)


def _make_pp_kernel(n, e):
    epc = e // (NC * NS)
    nchunks = epc // K

    @functools.partial(
        pl.kernel,
        out_type=jax.ShapeDtypeStruct((2 * n, 16), jnp.float32),
        mesh=_mesh(),
        compiler_params=pltpu.CompilerParams(needs_layout_passes=False,
                                             use_tc_tiling_on_sc=False),
        scratch_types=(
            [pltpu.VMEM((K,), jnp.int32) for _ in range(NB)]
            + [pltpu.VMEM((K, 16), jnp.float32) for _ in range(NB)]
            + [pltpu.VMEM_SHARED((n + 1, 16), jnp.float32)]
            + [pltpu.SemaphoreType.DMA for _ in range(2 * NB)]
        ),
    )
    def pp(dstm_h, eamf_h, z_h, out_h, *scr):
        dstmb = scr[0:NB]
        rowb = scr[NB:2 * NB]
        tab_sh = scr[2 * NB]
        s_ix = scr[2 * NB + 1:2 * NB + 1 + NB]
        s_sc = scr[2 * NB + 1 + NB:2 * NB + 1 + 2 * NB]
        c = lax.axis_index("c")
        s = lax.axis_index("s")
        _shard_rows(n, s, lambda o, l: pltpu.sync_copy(
            z_h.at[pl.ds(o, l)], tab_sh.at[pl.ds(o, l)]))
        plsc.subcore_barrier()
        base0 = (c * NS + s) * epc

        def issue_ix(i, b):
            pltpu.async_copy(dstm_h.at[pl.ds(base0 + i * K, K)], dstmb[b],
                             s_ix[b])
            pltpu.async_copy(eamf_h.at[pl.ds(base0 + i * K, K)], rowb[b],
                             s_ix[b])

        def drain_sc(b):
            pltpu.make_async_copy(rowb[b], tab_sh.at[dstmb[b]],
                                  s_sc[b]).wait()

        def body(i, b):
            p2 = (b + 2) % NB
            pltpu.make_async_copy(dstm_h.at[pl.ds(0, K)], dstmb[b],
                                  s_ix[b]).wait()
            pltpu.make_async_copy(eamf_h.at[pl.ds(0, K)], rowb[b],
                                  s_ix[b]).wait()
            pltpu.async_copy(rowb[b], tab_sh.at[dstmb[b]], s_sc[b], add=True)

            def _pref():
                _when(i >= 2, lambda: drain_sc(p2))
                issue_ix(i + 2, p2)
            _when(i + 2 < nchunks, _pref)

        issue_ix(0, 0)
        issue_ix(1, 1)

        def outer(g, carry):
            for t in range(NB):
                body(NB * g + t, t)
            return carry
        lax.fori_loop(0, nchunks // NB, outer, None)
        for t in range(nchunks % NB):
            body(nchunks - nchunks % NB + t, t)
        for j in range(max(nchunks - 4, 0), nchunks):
            drain_sc(j % NB)
        plsc.subcore_barrier()
        _shard_rows(n, s, lambda o, l: pltpu.sync_copy(
            tab_sh.at[pl.ds(o, l)], out_h.at[pl.ds(c * n + o, l)]))

    return pp


def _make_edge_kernel(n, e, hpc, w, split_edges):
    ev = 16 // hpc
    epc = e // (NC * NS) if split_edges else e // NS
    nchunks = epc // K
    r3 = 3 if w == 128 else 4
    r2 = 2 if w == 128 else 4
    UN = 12 if w == 128 else 4

    @functools.partial(
        pl.kernel,
        out_type=(jax.ShapeDtypeStruct((2 * n, w), jnp.float32),
                  jax.ShapeDtypeStruct((2 * n, 16), jnp.float32)),
        mesh=_mesh(),
        compiler_params=pltpu.CompilerParams(needs_layout_passes=False,
                                             use_tc_tiling_on_sc=False),
        scratch_types=(
            [pltpu.VMEM((K,), jnp.int32) for _ in range(r3)]
            + [pltpu.VMEM((K,), jnp.int32) for _ in range(4)]
            + [pltpu.VMEM((K,), jnp.int32) for _ in range(r3)]
            + [pltpu.VMEM((K, 16), jnp.float32) for _ in range(r2)]
            + [pltpu.VMEM((K, 16), jnp.float32) for _ in range(r2)]
            + [pltpu.VMEM((K, 16), jnp.float32) for _ in range(r2)]
            + [pltpu.VMEM((K, 8), jnp.float32) for _ in range(r2)]
            + [pltpu.VMEM((K, w), jnp.float32) for _ in range(r3)]
            + [pltpu.VMEM_SHARED((n + 1, w), jnp.float32),
               pltpu.VMEM_SHARED((n + 1, 16), jnp.float32)]
            + [pltpu.SemaphoreType.DMA for _ in range(4 + r3 + r2)]
        ),
    )
    def gat(src_h, dstm_h, aed_h, al_h, arm_h, hcat_h, inum_h, iden_h,
            num_o, den_o, *scr):
        o = 0
        srcb = scr[o:o + r3]; o += r3
        dstmb = scr[o:o + 4]; o += 4
        soffb = scr[o:o + r3]; o += r3
        bufA = scr[o:o + r2]; o += r2
        bufB = scr[o:o + r2]; o += r2
        pb = scr[o:o + r2]; o += r2
        aedb = scr[o:o + r2]; o += r2
        hb = scr[o:o + r3]; o += r3
        num_sh = scr[o]; o += 1
        den_sh = scr[o]; o += 1
        s_ix = scr[o:o + 4]; o += 4
        s_g = scr[o:o + r3]; o += r3
        s_sc = scr[o:o + r2]; o += r2

        c = lax.axis_index("c")
        s = lax.axis_index("s")

        def _init(o, l):
            pltpu.sync_copy(inum_h.at[pl.ds(c * n + o, l)],
                            num_sh.at[pl.ds(o, l)])
            pltpu.sync_copy(iden_h.at[pl.ds(c * n + o, l)],
                            den_sh.at[pl.ds(o, l)])
        _shard_rows(n, s, _init)
        plsc.subcore_barrier()
        base0 = (c * NS + s) * epc if split_edges else s * epc
        hoff = 0 if split_edges else c * n
        lane = lax.iota(jnp.int32, 16)
        erep = lax.shift_right_logical(lane, {1: 4, 2: 3, 4: 2, 8: 1, 16: 0}[ev])
        colidx = lax.rem(lane, hpc)

        def issue_ix(ti, t):
            pltpu.async_copy(src_h.at[pl.ds(base0 + ti * K, K)], srcb[t % r3],
                             s_ix[t % 4])
            pltpu.async_copy(dstm_h.at[pl.ds(base0 + ti * K, K)], dstmb[t % 4],
                             s_ix[t % 4])

        def issue_g(ti, t):
            pltpu.make_async_copy(src_h.at[pl.ds(0, K)], srcb[t % r3],
                                  s_ix[t % 4]).wait()
            pltpu.make_async_copy(dstm_h.at[pl.ds(0, K)], dstmb[t % 4],
                                  s_ix[t % 4]).wait()
            pltpu.async_copy(aed_h.at[pl.ds(base0 + ti * K, K)], aedb[t % r2],
                             s_g[t % r3])
            if split_edges:
                pltpu.async_copy(hcat_h.at[srcb[t % r3]], hb[t % r3], s_g[t % r3])
            else:
                def add16(v, cc):
                    sl = pl.ds(v * 16, 16)
                    soffb[t % r3][sl] = srcb[t % r3][sl] + hoff
                    return cc
                lax.fori_loop(0, K // 16, add16, None)
                pltpu.async_copy(hcat_h.at[soffb[t % r3]], hb[t % r3], s_g[t % r3])
            pltpu.async_copy(al_h.at[srcb[t % r3]], bufA[t % r2], s_g[t % r3])
            pltpu.async_copy(arm_h.at[dstmb[t % 4]], bufB[t % r2], s_g[t % r3])

        def drain_sc(t):
            pltpu.make_async_copy(pb[t % r2], den_sh.at[dstmb[t % 4]],
                                  s_sc[t % r2]).wait()
            pltpu.make_async_copy(hb[t % r3], num_sh.at[dstmb[t % 4]],
                                  s_sc[t % r2]).wait()

        def body(i, t):
            _when(i >= 2, lambda: drain_sc(t - 2))
            _when(i + 2 < nchunks, lambda: issue_ix(i + 2, t + 2))
            _when(i + 1 < nchunks, lambda: issue_g(i + 1, t + 1))

            pltpu.make_async_copy(hcat_h.at[srcb[t % r3]], hb[t % r3],
                                  s_g[t % r3]).wait()
            pltpu.make_async_copy(al_h.at[srcb[t % r3]], bufA[t % r2],
                                  s_g[t % r3]).wait()
            pltpu.make_async_copy(arm_h.at[dstmb[t % 4]], bufB[t % r2],
                                  s_g[t % r3]).wait()
            pltpu.make_async_copy(aed_h.at[pl.ds(0, K)], aedb[t % r2],
                                  s_g[t % r3]).wait()

            col = colidx if split_edges else 4 * c + colidx

            def pcomp(v, cc):
                rloc = erep + v * ev
                aA = plsc.load_gather(bufA[t % r2], [rloc, col])
                aB = plsc.load_gather(bufB[t % r2], [rloc, col])
                aE = plsc.load_gather(aedb[t % r2], [rloc, col])
                a = aA + aB + aE
                a = jnp.where(a >= 0.0, a, 0.2 * a)
                plsc.store_scatter(pb[t % r2], [rloc, colidx], jnp.exp(a))
                return cc
            lax.fori_loop(0, K // ev, pcomp, None)

            def mul(j, cc):
                jj = jnp.full((16,), j, jnp.int32)
                for hh in range(hpc):
                    colh = jnp.full((16,), hh, jnp.int32)
                    sp = plsc.load_gather(pb[t % r2], [jj, colh])
                    for qq in range(w // (16 * hpc)):
                        o = hh * (w // hpc) + qq * 16
                        hb[t % r3][j, pl.ds(o, 16)] = (
                            hb[t % r3][j, pl.ds(o, 16)] * sp)
                return cc
            lax.fori_loop(0, K, mul, None)

            pltpu.async_copy(pb[t % r2], den_sh.at[dstmb[t % 4]],
                             s_sc[t % r2], add=True)
            pltpu.async_copy(hb[t % r3], num_sh.at[dstmb[t % 4]],
                             s_sc[t % r2], add=True)

        issue_ix(0, 0)
        issue_ix(1, 1)
        issue_g(0, 0)

        def outer(g, carry):
            for t in range(UN):
                body(UN * g + t, t)
            return carry
        lax.fori_loop(0, nchunks // UN, outer, None)
        for t in range(nchunks % UN):
            body(nchunks - nchunks % UN + t, t)
        drain_sc(nchunks - 2)
        drain_sc(nchunks - 1)
        plsc.subcore_barrier()

        def _out(o, l):
            pltpu.sync_copy(num_sh.at[pl.ds(o, l)],
                            num_o.at[pl.ds(c * n + o, l)])
            pltpu.sync_copy(den_sh.at[pl.ds(o, l)],
                            den_o.at[pl.ds(c * n + o, l)])
        _shard_rows(n, s, _out)

    return gat


def _finalize_body(num_ref, den_ref, b_ref, o_ref, *, relu):
    v = num_ref[...] / (den_ref[...] + 1e-16) + b_ref[...]
    o_ref[...] = jnp.maximum(v, 0.0) if relu else v


def _finalize(num, den_full, b, relu):
    n, d = num.shape
    blk = 2000
    return pl.pallas_call(
        functools.partial(_finalize_body, relu=relu),
        grid=(n // blk,),
        in_specs=[
            pl.BlockSpec((blk, d), lambda i: (i, 0)),
            pl.BlockSpec((blk, d), lambda i: (i, 0)),
            pl.BlockSpec((1, d), lambda i: (0, 0)),
        ],
        out_specs=pl.BlockSpec((blk, d), lambda i: (i, 0)),
        out_shape=jax.ShapeDtypeStruct((n, d), num.dtype),
    )(num, den_full, b.reshape(1, d))


def _lrelu(a):
    return jnp.where(a >= 0, a, 0.2 * a)


def kernel(x, edge_index, edge_attr, W_se, b_se, W1, att_src1, att_dst1,
           W_e1, att_e1, b1, W2, att_src2, att_dst2, W_e2, att_e2, b2):
    n, d_in = x.shape
    heads, ch = att_src1.shape
    e = edge_index.shape[1]
    src, dst = edge_index[0], edge_index[1]
    dstm = jnp.where(src == dst, n, dst)

    ea = jax.nn.relu(edge_attr @ W_se + b_se)
    eamf = jnp.concatenate(
        [ea, jnp.ones((e, 1), jnp.float32), jnp.zeros((e, 7), jnp.float32)],
        axis=1)
    tab2 = _make_pp_kernel(n, e)(dstm, eamf, jnp.zeros((n, 16), jnp.float32))
    tab = tab2[:n] + tab2[n:]
    deg = tab[:, 8]
    loop_ea = tab[:, :8] / jnp.clip(deg, 1.0)[:, None]

    w1r = W1.reshape(d_in, heads, ch)
    ws1 = jnp.einsum('dhc,hc->dh', w1r, att_src1)
    wd1 = jnp.einsum('dhc,hc->dh', w1r, att_dst1)
    we1 = jnp.einsum('dhc,hc->dh', W_e1.reshape(-1, heads, ch), att_e1)
    h = x @ W1
    al = x @ ws1
    ar = x @ wd1
    aed_e = ea @ we1
    aed_n = loop_ea @ we1
    alA = jnp.tile(al, (1, 2))
    arm = jnp.concatenate(
        [jnp.tile(ar, (1, 2)), jnp.full((1, 16), -1e5, jnp.float32)], axis=0)
    hcat = h.reshape(n, 2, 128).transpose(1, 0, 2).reshape(2 * n, 128)
    p_self = jnp.exp(_lrelu(al + ar + aed_n))
    inum = (p_self[:, :, None] * h.reshape(n, heads, ch)).reshape(n, 256)
    inum = inum.reshape(n, 2, 128).transpose(1, 0, 2).reshape(2 * n, 128)
    iden = jnp.concatenate(
        [p_self.reshape(n, 2, 4).transpose(1, 0, 2).reshape(2 * n, 4),
         jnp.zeros((2 * n, 12), jnp.float32)], axis=1)

    num_o, den_o = _make_edge_kernel(n, e, 4, 128, False)(
        src, dstm, aed_e, alA, arm, hcat, inum, iden)
    num1 = num_o.reshape(2, n, 128).transpose(1, 0, 2).reshape(n, 256)
    den1 = jnp.concatenate([den_o[:n, :4], den_o[n:, :4]], axis=1)
    h1 = _finalize(num1, jnp.repeat(den1, ch, axis=1), b1, relu=True)

    ws2 = W2 @ att_src2[0]
    wd2 = W2 @ att_dst2[0]
    we2 = W_e2 @ att_e2[0]
    h2 = h1 @ W2
    al2 = h1 @ ws2
    ar2 = h1 @ wd2
    aed2_e = jnp.pad((ea @ we2)[:, None], ((0, 0), (0, 7)))
    aed2_n = loop_ea @ we2
    alA2 = jnp.pad(al2[:, None], ((0, 0), (0, 15)))
    arm2 = jnp.concatenate(
        [jnp.pad(ar2[:, None], ((0, 0), (0, 15))),
         jnp.full((1, 16), -1e5, jnp.float32)], axis=0)
    p2s = jnp.exp(_lrelu(al2 + ar2 + aed2_n))
    inum2 = jnp.concatenate(
        [p2s[:, None] * h2, jnp.zeros((n, 32), jnp.float32)], axis=0)
    iden2 = jnp.concatenate(
        [jnp.pad(p2s[:, None], ((0, 0), (0, 15))),
         jnp.zeros((n, 16), jnp.float32)], axis=0)

    num_o2, den_o2 = _make_edge_kernel(n, e, 1, 32, True)(
        src, dstm, aed2_e, alA2, arm2, h2, inum2, iden2)
    num2 = num_o2[:n] + num_o2[n:]
    den2 = den_o2[:n, :1] + den_o2[n:, :1]
    out = _finalize(num2, jnp.tile(den2, (1, 32)), b2, relu=False)
    return out
